# Initial kernel scaffold; baseline (speedup 1.0000x reference)
import jax
import jax.numpy as jnp
from jax.experimental import pallas as pl


def kernel(x, W):
    raise NotImplementedError("write your pallas kernel here")


# speedup vs baseline: 1.0362x; 1.0362x over previous
import functools

import jax
import jax.numpy as jnp
from jax import lax
from jax.experimental import pallas as pl
from jax.experimental.pallas import tpu as pltpu
from jax.experimental.pallas import tpu_sc as plsc

_TOPK = 8
_NE = 64
_NT = 8192
_D = 4096
_NC = 2
_NS = 16
_NW = _NC * _NS
_TPW = _NT // _NW
_L = 16
_GROUPS = _TPW // _L


def _logits_body(w_ref, x_ref, out_ref):
    out_ref[0] = lax.dot_general(
        w_ref[:], x_ref[:], (((1,), (1,)), ((), ())),
        preferred_element_type=jnp.float32)


def _compute_logits(x, W):
    return pl.pallas_call(
        _logits_body,
        grid=(_NW,),
        in_specs=[
            pl.BlockSpec((_NE, _D), lambda i: (0, 0)),
            pl.BlockSpec((_TPW, _D), lambda i: (i, 0)),
        ],
        out_specs=pl.BlockSpec((1, _NE, _TPW), lambda i: (i, 0, 0)),
        out_shape=jax.ShapeDtypeStruct((_NW, _NE, _TPW), jnp.float32),
    )(W, x)


def _make_topk():
    mesh = plsc.VectorSubcoreMesh(core_axis_name="c", subcore_axis_name="s")

    @functools.partial(
        pl.kernel, mesh=mesh,
        out_type=[
            jax.ShapeDtypeStruct((_NW, _TOPK, _TPW), jnp.float32),
            jax.ShapeDtypeStruct((_NW, _TOPK, _TPW), jnp.int32),
        ],
        scratch_types=[
            pltpu.VMEM((_NE, _TPW), jnp.float32),
            pltpu.VMEM((_TOPK, _TPW), jnp.float32),
            pltpu.VMEM((_TOPK, _TPW), jnp.int32),
        ],
    )
    def topk_kernel(l_hbm, w_hbm, i_hbm, lv, wv, iv):
        wid = lax.axis_index("s") * _NC + lax.axis_index("c")
        pltpu.sync_copy(l_hbm.at[wid], lv)

        def group_body(g, carry):
            base = g * _L
            neg_inf = jnp.full((_L,), -jnp.inf, jnp.float32)
            zero_i = jnp.zeros((_L,), jnp.int32)
            bv = [neg_inf] * _TOPK
            bi = [zero_i] * _TOPK
            for e in range(_NE):
                v = lv[e, pl.ds(base, _L)]
                ev = jnp.full((_L,), e, jnp.int32)
                c = [v > b for b in bv]
                nbv = [jnp.where(c[0], v, bv[0])]
                nbi = [jnp.where(c[0], ev, bi[0])]
                for i in range(1, _TOPK):
                    tv = jnp.where(c[i - 1], bv[i - 1], v)
                    ti = jnp.where(c[i - 1], bi[i - 1], ev)
                    nbv.append(jnp.where(c[i], tv, bv[i]))
                    nbi.append(jnp.where(c[i], ti, bi[i]))
                bv, bi = nbv, nbi
            m = bv[0]
            ex = [jnp.exp(b - m) for b in bv]
            s = ex[0]
            for k in range(1, _TOPK):
                s = s + ex[k]
            inv = 1.0 / s
            for k in range(_TOPK):
                wv[k, pl.ds(base, _L)] = ex[k] * inv
                iv[k, pl.ds(base, _L)] = bi[k]
            return carry

        lax.fori_loop(0, _GROUPS, group_body, 0)
        pltpu.sync_copy(wv, w_hbm.at[wid])
        pltpu.sync_copy(iv, i_hbm.at[wid])

    return topk_kernel


_topk = _make_topk()


def kernel(x, W):
    logits = _compute_logits(x, W)
    w_t, i_t = _topk(logits)
    weights = w_t.transpose(0, 2, 1).reshape(_NT, _TOPK)
    indices = i_t.transpose(0, 2, 1).reshape(_NT, _TOPK)
    return (weights, indices)
